# baseline (speedup 1.0000x reference)
import functools

import jax
import jax.numpy as jnp
from jax.experimental import pallas as pl


def _area(r):
    return (r[..., 2] - r[..., 0]) * (r[..., 3] - r[..., 1])


def _jaccard(a1, a2):
    x0 = jnp.maximum(a1[..., 0], a2[..., 0])
    y0 = jnp.maximum(a1[..., 1], a2[..., 1])
    x1 = jnp.minimum(a1[..., 2], a2[..., 2])
    y1 = jnp.minimum(a1[..., 3], a2[..., 3])
    m = ((x0 < x1) & (y0 < y1)).astype(jnp.float32)
    inter = (x1 - x0) * (y1 - y0) * m
    return inter / (_area(a1) + _area(a2) - inter)


def _xywh2rect(v):
    return jnp.stack([v[..., 0] - v[..., 2] / 2, v[..., 1] - v[..., 3] / 2,
                      v[..., 0] + v[..., 2] / 2, v[..., 1] + v[..., 3] / 2], axis=-1)


def _logit(x, e=0.001):
    x = jnp.where(x == 0, e, x)
    x = jnp.where(x == 1, 1 - e, x)
    return jnp.log(x / (1 - x))


def _log(x, e=0.001):
    return jnp.log(jnp.maximum(x, e))


def _bce(x, z):
    return jnp.maximum(x, 0.0) - x * z + jnp.log(1.0 + jnp.exp(-jnp.abs(x)))


def _partials_kernel(inf_ref, gb_ref, go_ref, gc_ref,
                     bn_ref, bc_ref, on_ref, oc_ref, cn_ref, cc_ref):
    step = pl.program_id(0)
    x = inf_ref[...]
    ib = x[:, 0:4]
    io = x[:, 4:5]
    ic = x[:, 5:85]
    gb = gb_ref[...]
    go = go_ref[...]
    gcode = gc_ref[...]

    selb = (jnp.sum((gb != -1.0).astype(jnp.float32), axis=1, keepdims=True)
            > 0.0).astype(jnp.float32)
    d = gb - ib
    box_num = jnp.sum(jnp.sum(d * d, axis=1, keepdims=True) * selb)
    box_cnt = jnp.sum(selb)

    selo = (go != -1.0).astype(jnp.float32)
    ow = jnp.where(go >= 1.0, 2.0, 0.5)
    obj_num = jnp.sum(_bce(io, go) * ow * selo)
    obj_cnt = jnp.sum(selo)

    selc = (gcode != -1.0).astype(jnp.float32)
    lane = jax.lax.broadcasted_iota(jnp.int32, ic.shape, 1)
    onehot = (lane == gcode.astype(jnp.int32)).astype(jnp.float32)
    sp = jnp.maximum(ic, 0.0) + jnp.log(1.0 + jnp.exp(-jnp.abs(ic)))
    row_bce = (jnp.sum(sp, axis=1, keepdims=True)
               - jnp.sum(ic * onehot, axis=1, keepdims=True))
    cls_num = jnp.sum(row_bce * selc)
    cls_cnt = jnp.sum(selc)

    for ref, val in ((bn_ref, box_num), (bc_ref, box_cnt),
                     (on_ref, obj_num), (oc_ref, obj_cnt),
                     (cn_ref, cls_num), (cc_ref, cls_cnt)):
        prev = jnp.where(step == 0, jnp.zeros((1, 1), jnp.float32), ref[...])
        ref[...] = prev + val


@functools.partial(jax.jit, static_argnames=("rows",))
def _scale_partials(inf2d, gb2, go2, gc2, rows=2048):
    m = inf2d.shape[0]
    grid = m // rows
    scalar = jax.ShapeDtypeStruct((1, 1), jnp.float32)
    outs = pl.pallas_call(
        _partials_kernel,
        grid=(grid,),
        in_specs=[
            pl.BlockSpec((rows, 85), lambda i: (i, 0)),
            pl.BlockSpec((rows, 4), lambda i: (i, 0)),
            pl.BlockSpec((rows, 1), lambda i: (i, 0)),
            pl.BlockSpec((rows, 1), lambda i: (i, 0)),
        ],
        out_specs=[pl.BlockSpec((1, 1), lambda i: (0, 0))] * 6,
        out_shape=[scalar] * 6,
    )(inf2d, gb2, go2, gc2)
    return outs


def kernel(inference_0, inference_1, inference_2, anchors, strides, targets):
    inference = [inference_0, inference_1, inference_2]
    iou_threshold, box_weight = 0.5, 5.0
    s, a = anchors.shape[0], anchors.shape[1]
    b, n, _ = targets.shape
    C = inference[0].shape[4] - 5

    tgt = targets.reshape(-1, 5)
    mask = ~jnp.all(tgt == -1, axis=1)
    batch_index = jnp.repeat(jnp.arange(b), n)
    anc = jnp.concatenate([jnp.full_like(anchors, 0.5),
                           anchors / strides[:, None, None]], axis=2)
    tb = jnp.repeat(tgt[:, None, 0:4], s, axis=1)
    tb = jnp.where(mask[:, None, None], tb / strides[None, :, None], tb)
    c = tb[..., 0:2].astype(jnp.int32)
    tb = tb.at[..., 0:2].set(jnp.where(mask[:, None, None],
                                       tb[..., 0:2] - c.astype(jnp.float32),
                                       tb[..., 0:2]))
    iou = _jaccard(_xywh2rect(tb).reshape(-1, s, 1, 4),
                   _xywh2rect(anc).reshape(1, s, a, 4))
    best = (jax.nn.one_hot(jnp.argmax(iou.reshape(-1, s * a), axis=1), s * a)
            > 0.5).reshape(-1, s, a)
    overlap = (iou > iou_threshold) & (~best)
    scale_mask = jnp.any(best, axis=2)
    anchor_index = jnp.argmax(jnp.any(best, axis=1).astype(jnp.float32), axis=1)
    tb = tb.at[..., 0:2].set(jnp.where(mask[:, None, None],
                                       _logit(tb[..., 0:2]), tb[..., 0:2]))
    tb4 = jnp.repeat(tb[:, :, None, :], a, axis=2)
    tb4 = tb4.at[..., 2:4].set(jnp.where(mask[:, None, None, None],
                                         _log(tb4[..., 2:4] / anc[None, :, :, 2:4]),
                                         tb4[..., 2:4]))
    tobj = best.astype(jnp.float32) - overlap.astype(jnp.float32)
    tcls = jax.nn.one_hot(jnp.clip(tgt[:, 4].astype(jnp.int32), 0, C - 1), C)
    tb_best = jnp.sum(tb4 * best[..., None].astype(jnp.float32), axis=(1, 2))

    totals = [jnp.float32(0.0)] * 6
    for i in range(s):
        H, W = inference[i].shape[2], inference[i].shape[3]
        m = mask & scale_mask[:, i]
        bi = jnp.where(m, batch_index, b)
        bi2 = jnp.where(mask, batch_index, b)
        yy = c[:, i, 1]
        xx = c[:, i, 0]
        gb = jnp.full((b, a, H, W, 4), -1.0, dtype=jnp.float32
                      ).at[bi, anchor_index, yy, xx].set(tb_best, mode='drop')
        go = jnp.zeros((b, H, W, a), dtype=jnp.float32
                       ).at[bi2, yy, xx].set(tobj[:, i, :], mode='drop'
                                             ).transpose(0, 3, 1, 2)
        m2 = (overlap[:, i] | best[:, i]).astype(jnp.float32)
        cls_f = jnp.clip(tgt[:, 4].astype(jnp.int32), 0, C - 1).astype(jnp.float32)
        codes = jnp.where(m2 > 0.0, cls_f[:, None], -1.0)
        gc = jnp.full((b, H, W, a), -1.0, dtype=jnp.float32
                      ).at[bi2, yy, xx].set(codes, mode='drop'
                                            ).transpose(0, 3, 1, 2)
        mi = b * a * H * W
        outs = _scale_partials(inference[i].reshape(mi, 5 + C),
                               gb.reshape(mi, 4),
                               go.reshape(mi, 1),
                               gc.reshape(mi, 1))
        totals = [t + o[0, 0] for t, o in zip(totals, outs)]

    box_num, box_cnt, obj_num, obj_cnt, cls_num, cls_cnt = totals
    box_loss = box_weight * box_num / jnp.maximum(box_cnt, 1.0)
    obj_loss = obj_num / jnp.maximum(obj_cnt, 1.0)
    cls_loss = cls_num / jnp.maximum(cls_cnt * C, 1.0)
    return box_loss + obj_loss + cls_loss


# speedup vs baseline: 1.0811x; 1.0811x over previous
import functools

import jax
import jax.numpy as jnp
from jax.experimental import pallas as pl


def _area(r):
    return (r[..., 2] - r[..., 0]) * (r[..., 3] - r[..., 1])


def _jaccard(a1, a2):
    x0 = jnp.maximum(a1[..., 0], a2[..., 0])
    y0 = jnp.maximum(a1[..., 1], a2[..., 1])
    x1 = jnp.minimum(a1[..., 2], a2[..., 2])
    y1 = jnp.minimum(a1[..., 3], a2[..., 3])
    m = ((x0 < x1) & (y0 < y1)).astype(jnp.float32)
    inter = (x1 - x0) * (y1 - y0) * m
    return inter / (_area(a1) + _area(a2) - inter)


def _xywh2rect(v):
    return jnp.stack([v[..., 0] - v[..., 2] / 2, v[..., 1] - v[..., 3] / 2,
                      v[..., 0] + v[..., 2] / 2, v[..., 1] + v[..., 3] / 2], axis=-1)


def _logit(x, e=0.001):
    x = jnp.where(x == 0, e, x)
    x = jnp.where(x == 1, 1 - e, x)
    return jnp.log(x / (1 - x))


def _log(x, e=0.001):
    return jnp.log(jnp.maximum(x, e))


def _bce(x, z):
    return jnp.maximum(x, 0.0) - x * z + jnp.log(1.0 + jnp.exp(-jnp.abs(x)))


def _partials_kernel(inf_ref, gb_ref, go_ref, gc_ref,
                     bn_ref, bc_ref, on_ref, oc_ref, cn_ref, cc_ref):
    step = pl.program_id(0)
    x = inf_ref[...]
    ib = x[:, 0:4]
    io = x[:, 4:5]
    ic = x[:, 5:85]
    gb = gb_ref[...]
    go = go_ref[...]
    gcode = gc_ref[...]

    selb = (jnp.sum((gb != -1.0).astype(jnp.float32), axis=1, keepdims=True)
            > 0.0).astype(jnp.float32)
    d = gb - ib
    box_num = jnp.sum(jnp.sum(d * d, axis=1, keepdims=True) * selb)
    box_cnt = jnp.sum(selb)

    selo = (go != -1.0).astype(jnp.float32)
    ow = jnp.where(go >= 1.0, 2.0, 0.5)
    obj_num = jnp.sum(_bce(io, go) * ow * selo)
    obj_cnt = jnp.sum(selo)

    selc = (gcode != -1.0).astype(jnp.float32)
    lane = jax.lax.broadcasted_iota(jnp.int32, ic.shape, 1)
    onehot = (lane == gcode.astype(jnp.int32)).astype(jnp.float32)
    sp = jnp.maximum(ic, 0.0) + jnp.log(1.0 + jnp.exp(-jnp.abs(ic)))
    row_bce = (jnp.sum(sp, axis=1, keepdims=True)
               - jnp.sum(ic * onehot, axis=1, keepdims=True))
    cls_num = jnp.sum(row_bce * selc)
    cls_cnt = jnp.sum(selc)

    for ref, val in ((bn_ref, box_num), (bc_ref, box_cnt),
                     (on_ref, obj_num), (oc_ref, obj_cnt),
                     (cn_ref, cls_num), (cc_ref, cls_cnt)):
        prev = jnp.where(step == 0, jnp.zeros((1, 1), jnp.float32), ref[...])
        ref[...] = prev + val


@functools.partial(jax.jit, static_argnames=("rows",))
def _scale_partials(inf2d, gb2, go2, gc2, rows=2048):
    m = inf2d.shape[0]
    rows = min(rows, m)
    grid = m // rows
    scalar = jax.ShapeDtypeStruct((1, 1), jnp.float32)
    outs = pl.pallas_call(
        _partials_kernel,
        grid=(grid,),
        in_specs=[
            pl.BlockSpec((rows, 85), lambda i: (i, 0)),
            pl.BlockSpec((rows, 4), lambda i: (i, 0)),
            pl.BlockSpec((rows, 1), lambda i: (i, 0)),
            pl.BlockSpec((rows, 1), lambda i: (i, 0)),
        ],
        out_specs=[pl.BlockSpec((1, 1), lambda i: (0, 0))] * 6,
        out_shape=[scalar] * 6,
    )(inf2d, gb2, go2, gc2)
    return outs


def kernel(inference_0, inference_1, inference_2, anchors, strides, targets):
    inference = [inference_0, inference_1, inference_2]
    iou_threshold, box_weight = 0.5, 5.0
    s, a = anchors.shape[0], anchors.shape[1]
    b, n, _ = targets.shape
    C = inference[0].shape[4] - 5

    tgt = targets.reshape(-1, 5)
    mask = ~jnp.all(tgt == -1, axis=1)
    batch_index = jnp.repeat(jnp.arange(b), n)
    anc = jnp.concatenate([jnp.full_like(anchors, 0.5),
                           anchors / strides[:, None, None]], axis=2)
    tb = jnp.repeat(tgt[:, None, 0:4], s, axis=1)
    tb = jnp.where(mask[:, None, None], tb / strides[None, :, None], tb)
    c = tb[..., 0:2].astype(jnp.int32)
    tb = tb.at[..., 0:2].set(jnp.where(mask[:, None, None],
                                       tb[..., 0:2] - c.astype(jnp.float32),
                                       tb[..., 0:2]))
    iou = _jaccard(_xywh2rect(tb).reshape(-1, s, 1, 4),
                   _xywh2rect(anc).reshape(1, s, a, 4))
    best = (jax.nn.one_hot(jnp.argmax(iou.reshape(-1, s * a), axis=1), s * a)
            > 0.5).reshape(-1, s, a)
    overlap = (iou > iou_threshold) & (~best)
    scale_mask = jnp.any(best, axis=2)
    anchor_index = jnp.argmax(jnp.any(best, axis=1).astype(jnp.float32), axis=1)
    tb = tb.at[..., 0:2].set(jnp.where(mask[:, None, None],
                                       _logit(tb[..., 0:2]), tb[..., 0:2]))
    tb4 = jnp.repeat(tb[:, :, None, :], a, axis=2)
    tb4 = tb4.at[..., 2:4].set(jnp.where(mask[:, None, None, None],
                                         _log(tb4[..., 2:4] / anc[None, :, :, 2:4]),
                                         tb4[..., 2:4]))
    tobj = best.astype(jnp.float32) - overlap.astype(jnp.float32)
    tcls = jax.nn.one_hot(jnp.clip(tgt[:, 4].astype(jnp.int32), 0, C - 1), C)
    tb_best = jnp.sum(tb4 * best[..., None].astype(jnp.float32), axis=(1, 2))

    totals = [jnp.float32(0.0)] * 6
    for i in range(s):
        H, W = inference[i].shape[2], inference[i].shape[3]
        m = mask & scale_mask[:, i]
        bi = jnp.where(m, batch_index, b)
        bi2 = jnp.where(mask, batch_index, b)
        yy = c[:, i, 1]
        xx = c[:, i, 0]
        gb = jnp.full((b, a, H, W, 4), -1.0, dtype=jnp.float32
                      ).at[bi, anchor_index, yy, xx].set(tb_best, mode='drop')
        go = jnp.zeros((b, H, W, a), dtype=jnp.float32
                       ).at[bi2, yy, xx].set(tobj[:, i, :], mode='drop'
                                             ).transpose(0, 3, 1, 2)
        m2 = (overlap[:, i] | best[:, i]).astype(jnp.float32)
        cls_f = jnp.clip(tgt[:, 4].astype(jnp.int32), 0, C - 1).astype(jnp.float32)
        codes = jnp.where(m2 > 0.0, cls_f[:, None], -1.0)
        gc = jnp.full((b, H, W, a), -1.0, dtype=jnp.float32
                      ).at[bi2, yy, xx].set(codes, mode='drop'
                                            ).transpose(0, 3, 1, 2)
        mi = b * a * H * W
        outs = _scale_partials(inference[i].reshape(mi, 5 + C),
                               gb.reshape(mi, 4),
                               go.reshape(mi, 1),
                               gc.reshape(mi, 1), rows=8192)
        totals = [t + o[0, 0] for t, o in zip(totals, outs)]

    box_num, box_cnt, obj_num, obj_cnt, cls_num, cls_cnt = totals
    box_loss = box_weight * box_num / jnp.maximum(box_cnt, 1.0)
    obj_loss = obj_num / jnp.maximum(obj_cnt, 1.0)
    cls_loss = cls_num / jnp.maximum(cls_cnt * C, 1.0)
    return box_loss + obj_loss + cls_loss


# speedup vs baseline: 1.1984x; 1.1085x over previous
import functools

import jax
import jax.numpy as jnp
from jax.experimental import pallas as pl


def _area(r):
    return (r[..., 2] - r[..., 0]) * (r[..., 3] - r[..., 1])


def _jaccard(a1, a2):
    x0 = jnp.maximum(a1[..., 0], a2[..., 0])
    y0 = jnp.maximum(a1[..., 1], a2[..., 1])
    x1 = jnp.minimum(a1[..., 2], a2[..., 2])
    y1 = jnp.minimum(a1[..., 3], a2[..., 3])
    m = ((x0 < x1) & (y0 < y1)).astype(jnp.float32)
    inter = (x1 - x0) * (y1 - y0) * m
    return inter / (_area(a1) + _area(a2) - inter)


def _xywh2rect(v):
    return jnp.stack([v[..., 0] - v[..., 2] / 2, v[..., 1] - v[..., 3] / 2,
                      v[..., 0] + v[..., 2] / 2, v[..., 1] + v[..., 3] / 2], axis=-1)


def _logit(x, e=0.001):
    x = jnp.where(x == 0, e, x)
    x = jnp.where(x == 1, 1 - e, x)
    return jnp.log(x / (1 - x))


def _log(x, e=0.001):
    return jnp.log(jnp.maximum(x, e))


def _bce(x, z):
    return jnp.maximum(x, 0.0) - x * z + jnp.log(1.0 + jnp.exp(-jnp.abs(x)))


def _partials_kernel(inf_ref, gb_ref, go_ref, gc_ref,
                     bn_ref, bc_ref, on_ref, oc_ref, cn_ref, cc_ref):
    step = pl.program_id(0)
    x = inf_ref[...]
    ib = x[:, 0:4]
    io = x[:, 4:5]
    ic = x[:, 5:85]
    gb = gb_ref[...]
    go = go_ref[...]
    gc = gc_ref[...]

    selb = (jnp.sum((gb != -1.0).astype(jnp.float32), axis=1, keepdims=True)
            > 0.0).astype(jnp.float32)
    d = gb - ib
    box_num = jnp.sum(jnp.sum(d * d, axis=1, keepdims=True) * selb)
    box_cnt = jnp.sum(selb)

    selo = (go != -1.0).astype(jnp.float32)
    ow = jnp.where(go >= 1.0, 2.0, 0.5)
    obj_num = jnp.sum(_bce(io, go) * ow * selo)
    obj_cnt = jnp.sum(selo)

    selc = (jnp.sum((gc == -1.0).astype(jnp.float32), axis=1, keepdims=True)
            == 0.0).astype(jnp.float32)
    cls_num = jnp.sum(_bce(ic, gc) * selc)
    cls_cnt = jnp.sum(selc)

    for ref, val in ((bn_ref, box_num), (bc_ref, box_cnt),
                     (on_ref, obj_num), (oc_ref, obj_cnt),
                     (cn_ref, cls_num), (cc_ref, cls_cnt)):
        prev = jnp.where(step == 0, jnp.zeros((1, 1), jnp.float32), ref[...])
        ref[...] = prev + val


@functools.partial(jax.jit, static_argnames=("rows",))
def _scale_partials(inf2d, gb2, go2, gc2, rows=2048):
    m = inf2d.shape[0]
    rows = min(rows, m)
    grid = m // rows
    scalar = jax.ShapeDtypeStruct((1, 1), jnp.float32)
    outs = pl.pallas_call(
        _partials_kernel,
        grid=(grid,),
        in_specs=[
            pl.BlockSpec((rows, 85), lambda i: (i, 0)),
            pl.BlockSpec((rows, 4), lambda i: (i, 0)),
            pl.BlockSpec((rows, 1), lambda i: (i, 0)),
            pl.BlockSpec((rows, 80), lambda i: (i, 0)),
        ],
        out_specs=[pl.BlockSpec((1, 1), lambda i: (0, 0))] * 6,
        out_shape=[scalar] * 6,
    )(inf2d, gb2, go2, gc2)
    return outs


def kernel(inference_0, inference_1, inference_2, anchors, strides, targets):
    inference = [inference_0, inference_1, inference_2]
    iou_threshold, box_weight = 0.5, 5.0
    s, a = anchors.shape[0], anchors.shape[1]
    b, n, _ = targets.shape
    C = inference[0].shape[4] - 5

    tgt = targets.reshape(-1, 5)
    mask = ~jnp.all(tgt == -1, axis=1)
    batch_index = jnp.repeat(jnp.arange(b), n)
    anc = jnp.concatenate([jnp.full_like(anchors, 0.5),
                           anchors / strides[:, None, None]], axis=2)
    tb = jnp.repeat(tgt[:, None, 0:4], s, axis=1)
    tb = jnp.where(mask[:, None, None], tb / strides[None, :, None], tb)
    c = tb[..., 0:2].astype(jnp.int32)
    tb = tb.at[..., 0:2].set(jnp.where(mask[:, None, None],
                                       tb[..., 0:2] - c.astype(jnp.float32),
                                       tb[..., 0:2]))
    iou = _jaccard(_xywh2rect(tb).reshape(-1, s, 1, 4),
                   _xywh2rect(anc).reshape(1, s, a, 4))
    best = (jax.nn.one_hot(jnp.argmax(iou.reshape(-1, s * a), axis=1), s * a)
            > 0.5).reshape(-1, s, a)
    overlap = (iou > iou_threshold) & (~best)
    scale_mask = jnp.any(best, axis=2)
    anchor_index = jnp.argmax(jnp.any(best, axis=1).astype(jnp.float32), axis=1)
    tb = tb.at[..., 0:2].set(jnp.where(mask[:, None, None],
                                       _logit(tb[..., 0:2]), tb[..., 0:2]))
    tb4 = jnp.repeat(tb[:, :, None, :], a, axis=2)
    tb4 = tb4.at[..., 2:4].set(jnp.where(mask[:, None, None, None],
                                         _log(tb4[..., 2:4] / anc[None, :, :, 2:4]),
                                         tb4[..., 2:4]))
    tobj = best.astype(jnp.float32) - overlap.astype(jnp.float32)
    tcls = jax.nn.one_hot(jnp.clip(tgt[:, 4].astype(jnp.int32), 0, C - 1), C)
    tb_best = jnp.sum(tb4 * best[..., None].astype(jnp.float32), axis=(1, 2))

    totals = [jnp.float32(0.0)] * 6
    for i in range(s):
        H, W = inference[i].shape[2], inference[i].shape[3]
        m = mask & scale_mask[:, i]
        bi = jnp.where(m, batch_index, b)
        bi2 = jnp.where(mask, batch_index, b)
        yy = c[:, i, 1]
        xx = c[:, i, 0]
        gb = jnp.full((b, a, H, W, 4), -1.0, dtype=jnp.float32
                      ).at[bi, anchor_index, yy, xx].set(tb_best, mode='drop')
        go = jnp.zeros((b, H, W, a), dtype=jnp.float32
                       ).at[bi2, yy, xx].set(tobj[:, i, :], mode='drop'
                                             ).transpose(0, 3, 1, 2)
        m2 = (overlap[:, i] | best[:, i]).astype(jnp.float32)
        vals = tcls[:, None, :] * m2[:, :, None] - (1.0 - m2[:, :, None])
        gc = jnp.full((b, H, W, a, C), -1.0, dtype=jnp.float32
                      ).at[bi2, yy, xx].set(vals, mode='drop'
                                            ).transpose(0, 3, 1, 2, 4)
        mi = b * a * H * W
        outs = _scale_partials(inference[i].reshape(mi, 5 + C),
                               gb.reshape(mi, 4),
                               go.reshape(mi, 1),
                               gc.reshape(mi, C), rows=8192)
        totals = [t + o[0, 0] for t, o in zip(totals, outs)]

    box_num, box_cnt, obj_num, obj_cnt, cls_num, cls_cnt = totals
    box_loss = box_weight * box_num / jnp.maximum(box_cnt, 1.0)
    obj_loss = obj_num / jnp.maximum(obj_cnt, 1.0)
    cls_loss = cls_num / jnp.maximum(cls_cnt * C, 1.0)
    return box_loss + obj_loss + cls_loss
